# baseline (speedup 1.0000x reference)
import dataclasses
import functools

import jax
import jax.numpy as jnp
from jax import lax
from jax.experimental import pallas as pl
from jax.experimental.pallas import tpu as pltpu
from jax.experimental.pallas import tpu_sc as plsc

N = 10000
E = 320000
D = 128
H = 128
L = 100000

NC = 2
NS = 16
NW = NC * NS

AW = 80
NWIN = 128
E2 = NW * NWIN * AW
NPAD = 10240
NROW = NPAD // NS
SBUF = 2
NBUF = 4

LPAD = 100352
GW = 128
GWIN = (2 * LPAD) // NW // GW

_f32 = jnp.float32
_PH = lax.Precision.HIGHEST
_MESH = plsc.VectorSubcoreMesh(core_axis_name="c", subcore_axis_name="s")

_CP = pltpu.CompilerParams()
if "needs_layout_passes" in pltpu.CompilerParams.__dataclass_fields__:
    _CP = dataclasses.replace(_CP, needs_layout_passes=False)


def _dot(a, b):
    return lax.dot(a, b, precision=_PH, preferred_element_type=_f32)


@functools.partial(
    pl.kernel, out_type=jax.ShapeDtypeStruct((NW, NPAD), _f32), mesh=_MESH,
    compiler_params=_CP,
    scratch_types=[pltpu.VMEM((NWIN, AW), jnp.int32),
                   pltpu.VMEM((NPAD,), _f32)])
def _deg_hist(dst_hbm, zdeg_hbm, pdeg_hbm, didx, hist):
    cid = lax.axis_index("c")
    sid = lax.axis_index("s")
    wid = sid * NC + cid
    pltpu.sync_copy(dst_hbm.at[wid], didx)
    pltpu.sync_copy(zdeg_hbm, hist)
    ones16 = jnp.full((16,), 1.0, _f32)

    @pl.loop(0, NWIN)
    def _(j):
        for t in range(AW // 16):
            plsc.addupdate_scatter(hist, [didx[j, pl.ds(t * 16, 16)]], ones16)

    pltpu.sync_copy(hist, pdeg_hbm.at[wid])


_SS_SCRATCH = ([pltpu.VMEM((NWIN, AW), jnp.int32)]
               + [pltpu.VMEM((AW,), jnp.int32) for _ in range(SBUF)]
               + [pltpu.VMEM((AW, H), _f32) for _ in range(SBUF)]
               + [pltpu.VMEM_SHARED((NPAD, H), _f32)]
               + [pltpu.SemaphoreType.DMA for _ in range(3 * SBUF)])


@functools.partial(
    pl.kernel, out_type=jax.ShapeDtypeStruct((NC, NPAD, H), _f32), mesh=_MESH,
    scratch_types=_SS_SCRATCH)
def _seg_sum(h_hbm, src_hbm, dst_hbm, zrow_hbm, psum_hbm, *refs):
    sidx = refs[0]
    dbufs = refs[1:1 + SBUF]
    rbufs = refs[1 + SBUF:1 + 2 * SBUF]
    acc = refs[1 + 2 * SBUF]
    sems = refs[2 + 2 * SBUF:]

    cid = lax.axis_index("c")
    sid = lax.axis_index("s")
    wid = sid * NC + cid
    r0 = sid * NROW
    pltpu.sync_copy(zrow_hbm, acc.at[pl.ds(r0, NROW)])
    pltpu.sync_copy(src_hbm.at[wid], sidx)
    plsc.subcore_barrier()

    @pl.loop(0, NWIN)
    def _(j):
        pltpu.sync_copy(dst_hbm.at[wid, j], dbufs[0])
        pltpu.sync_copy(h_hbm.at[sidx.at[j]], rbufs[0])
        pltpu.sync_copy(rbufs[0], acc.at[dbufs[0]], add=True)

    plsc.subcore_barrier()
    pltpu.sync_copy(acc.at[pl.ds(r0, NROW)], psum_hbm.at[cid, pl.ds(r0, NROW)])


@functools.partial(
    pl.kernel, out_type=jax.ShapeDtypeStruct((2 * LPAD, H), _f32), mesh=_MESH,
    scratch_types=([pltpu.VMEM((GWIN, GW), jnp.int32)]
                   + [pltpu.VMEM((GW, H), _f32) for _ in range(NBUF)]
                   + [pltpu.SemaphoreType.DMA for _ in range(2 * NBUF)]))
def _gather_rows(h_hbm, gidx_hbm, out_hbm, gidx, *refs):
    rbufs = refs[:NBUF]
    sems = refs[NBUF:]
    cid = lax.axis_index("c")
    sid = lax.axis_index("s")
    wid = sid * NC + cid
    base = wid * (GWIN * GW)
    pltpu.sync_copy(gidx_hbm.at[wid], gidx)
    main = (GWIN // NBUF) * NBUF

    @pl.loop(0, main, step=NBUF)
    def _(j):
        cgs = [pltpu.async_copy(h_hbm.at[gidx.at[j + b]], rbufs[b], sems[b])
               for b in range(NBUF)]
        cws = []
        for b in range(NBUF):
            cgs[b].wait()
            cws.append(pltpu.async_copy(
                rbufs[b], out_hbm.at[pl.ds(base + (j + b) * GW, GW)],
                sems[NBUF + b]))
        for b in range(NBUF):
            cws[b].wait()

    for j in range(main, GWIN):
        pltpu.sync_copy(h_hbm.at[gidx.at[j]], rbufs[0])
        pltpu.sync_copy(rbufs[0], out_hbm.at[pl.ds(base + j * GW, GW)])


def _sage_layer(x, psum, pdeg, Ws, Wn, b, relu):
    BN = 1024

    def body(x_ref, p_ref, d_ref, ws_ref, wn_ref, b_ref, o_ref):
        p = p_ref[...]
        deg = jnp.maximum(jnp.sum(d_ref[...], axis=0), 1.0)[:, None]
        agg = (p[0] + p[1]) / deg
        y = _dot(x_ref[...], ws_ref[...]) + _dot(agg, wn_ref[...]) + b_ref[...]
        if relu:
            y = jnp.maximum(y, 0.0)
        o_ref[...] = y + x_ref[...]

    return pl.pallas_call(
        body,
        grid=(NPAD // BN,),
        in_specs=[
            pl.BlockSpec((BN, H), lambda i: (i, 0)),
            pl.BlockSpec((NC, BN, H), lambda i: (0, i, 0)),
            pl.BlockSpec((NW, BN), lambda i: (0, i)),
            pl.BlockSpec((D, H), lambda i: (0, 0)),
            pl.BlockSpec((D, H), lambda i: (0, 0)),
            pl.BlockSpec((1, H), lambda i: (0, 0)),
        ],
        out_specs=pl.BlockSpec((BN, H), lambda i: (i, 0)),
        out_shape=jax.ShapeDtypeStruct((N, H), _f32),
    )(x, psum, pdeg, Ws, Wn, b.reshape(1, H))


def _mlp_head(rows, W1, b1, W2, b2):
    BL = 512
    nblk = LPAD // BL

    def body(hs_ref, hd_ref, w1a_ref, w1b_ref, b1_ref, w2_ref, b2_ref, o_ref):
        z = _dot(hs_ref[...], w1a_ref[...]) + _dot(hd_ref[...], w1b_ref[...]) + b1_ref[...]
        z = jnp.maximum(z, 0.0)
        o_ref[...] = _dot(z, w2_ref[...]) + b2_ref[...]

    return pl.pallas_call(
        body,
        grid=(nblk,),
        in_specs=[
            pl.BlockSpec((BL, H), lambda i: (i, 0)),
            pl.BlockSpec((BL, H), lambda i, _n=nblk: (i + _n, 0)),
            pl.BlockSpec((H, H), lambda i: (0, 0)),
            pl.BlockSpec((H, H), lambda i: (0, 0)),
            pl.BlockSpec((1, H), lambda i: (0, 0)),
            pl.BlockSpec((H, 1), lambda i: (0, 0)),
            pl.BlockSpec((1, 1), lambda i: (0, 0)),
        ],
        out_specs=pl.BlockSpec((BL, 1), lambda i: (i, 0)),
        out_shape=jax.ShapeDtypeStruct((LPAD, 1), _f32),
    )(rows, rows, W1[:H], W1[H:], b1.reshape(1, H), W2, b2.reshape(1, 1))


def kernel(x, edge_index, edge_label_index, W_self_0, W_neigh_0, bias_0,
           W_self_1, W_neigh_1, bias_1, mlp_W1, mlp_b1, mlp_W2, mlp_b2):
    ep = E2 - E
    srcp = jnp.concatenate([edge_index[0], jnp.zeros((ep,), jnp.int32)])
    dstp = jnp.concatenate(
        [edge_index[1], N + (jnp.arange(ep, dtype=jnp.int32) % 128)])
    src3 = srcp.reshape(NW, NWIN, AW)
    dst3 = dstp.reshape(NW, NWIN, AW)
    zrow = jnp.zeros((NROW, H), _f32)
    zdeg = jnp.zeros((NPAD,), _f32)

    pdeg = _deg_hist(dst3, zdeg)
    psum0 = _seg_sum(x, src3, dst3, zrow)
    h1 = _sage_layer(x, psum0, pdeg, W_self_0, W_neigh_0, bias_0, relu=True)
    psum1 = _seg_sum(h1, src3, dst3, zrow)
    h2 = _sage_layer(h1, psum1, pdeg, W_self_1, W_neigh_1, bias_1, relu=False)

    pad = jnp.zeros((LPAD - L,), jnp.int32)
    gidx3 = jnp.concatenate(
        [edge_label_index[0], pad, edge_label_index[1], pad]).reshape(NW, GWIN, GW)
    rows = _gather_rows(h2, gidx3)
    out = _mlp_head(rows, mlp_W1, mlp_b1, mlp_W2, mlp_b2)
    return out[:L, 0]


# speedup vs baseline: 1.0009x; 1.0009x over previous
import dataclasses
import functools

import jax
import jax.numpy as jnp
from jax import lax
from jax.experimental import pallas as pl
from jax.experimental.pallas import tpu as pltpu
from jax.experimental.pallas import tpu_sc as plsc

N = 10000
E = 320000
D = 128
H = 128
L = 100000

NC = 2
NS = 16
NW = NC * NS

AW = 80
NWIN = 128
E2 = NW * NWIN * AW
NPAD = 10240
NROW = NPAD // NS
SBUF = 2
NBUF = 4

LPAD = 100352
GW = 128
GWIN = (2 * LPAD) // NW // GW

_f32 = jnp.float32
_PH = lax.Precision.HIGHEST
_MESH = plsc.VectorSubcoreMesh(core_axis_name="c", subcore_axis_name="s")

_CP = pltpu.CompilerParams()
if "needs_layout_passes" in pltpu.CompilerParams.__dataclass_fields__:
    _CP = dataclasses.replace(_CP, needs_layout_passes=False)


def _dot(a, b):
    return lax.dot(a, b, precision=_PH, preferred_element_type=_f32)


@functools.partial(
    pl.kernel, out_type=jax.ShapeDtypeStruct((NW, NPAD), _f32), mesh=_MESH,
    compiler_params=_CP,
    scratch_types=[pltpu.VMEM((NWIN, AW), jnp.int32),
                   pltpu.VMEM((NPAD,), _f32)])
def _deg_hist(dst_hbm, zdeg_hbm, pdeg_hbm, didx, hist):
    cid = lax.axis_index("c")
    sid = lax.axis_index("s")
    wid = sid * NC + cid
    pltpu.sync_copy(dst_hbm.at[wid], didx)
    pltpu.sync_copy(zdeg_hbm, hist)
    ones16 = jnp.full((16,), 1.0, _f32)

    @pl.loop(0, NWIN)
    def _(j):
        for t in range(AW // 16):
            plsc.addupdate_scatter(hist, [didx[j, pl.ds(t * 16, 16)]], ones16)

    pltpu.sync_copy(hist, pdeg_hbm.at[wid])


_SS_SCRATCH = ([pltpu.VMEM((NWIN, AW), jnp.int32)]
               + [pltpu.VMEM((AW,), jnp.int32)]
               + [pltpu.VMEM((AW, H), _f32)]
               + [pltpu.VMEM_SHARED((NPAD, H), _f32)])


@functools.partial(
    pl.kernel, out_type=jax.ShapeDtypeStruct((NC, NPAD, H), _f32), mesh=_MESH,
    scratch_types=_SS_SCRATCH)
def _seg_sum(h_hbm, src_hbm, dst_hbm, zrow_hbm, psum_hbm, *refs):
    sidx = refs[0]
    dbufs = refs[1:2]
    rbufs = refs[2:3]
    acc = refs[3]

    cid = lax.axis_index("c")
    sid = lax.axis_index("s")
    wid = sid * NC + cid
    r0 = sid * NROW
    pltpu.sync_copy(zrow_hbm, acc.at[pl.ds(r0, NROW)])
    pltpu.sync_copy(src_hbm.at[wid], sidx)
    plsc.subcore_barrier()

    @pl.loop(0, NWIN)
    def _(j):
        pltpu.sync_copy(dst_hbm.at[wid, j], dbufs[0])
        pltpu.sync_copy(h_hbm.at[sidx.at[j]], rbufs[0])
        pltpu.sync_copy(rbufs[0], acc.at[dbufs[0]], add=True)

    plsc.subcore_barrier()
    pltpu.sync_copy(acc.at[pl.ds(r0, NROW)], psum_hbm.at[cid, pl.ds(r0, NROW)])


@functools.partial(
    pl.kernel, out_type=jax.ShapeDtypeStruct((2 * LPAD, H), _f32), mesh=_MESH,
    scratch_types=([pltpu.VMEM((GWIN, GW), jnp.int32)]
                   + [pltpu.VMEM((GW, H), _f32) for _ in range(NBUF)]
                   + [pltpu.SemaphoreType.DMA for _ in range(2 * NBUF)]))
def _gather_rows(h_hbm, gidx_hbm, out_hbm, gidx, *refs):
    rbufs = refs[:NBUF]
    sems = refs[NBUF:]
    cid = lax.axis_index("c")
    sid = lax.axis_index("s")
    wid = sid * NC + cid
    base = wid * (GWIN * GW)
    pltpu.sync_copy(gidx_hbm.at[wid], gidx)
    main = (GWIN // NBUF) * NBUF

    @pl.loop(0, main, step=NBUF)
    def _(j):
        cgs = [pltpu.async_copy(h_hbm.at[gidx.at[j + b]], rbufs[b], sems[b])
               for b in range(NBUF)]
        cws = []
        for b in range(NBUF):
            cgs[b].wait()
            cws.append(pltpu.async_copy(
                rbufs[b], out_hbm.at[pl.ds(base + (j + b) * GW, GW)],
                sems[NBUF + b]))
        for b in range(NBUF):
            cws[b].wait()

    for j in range(main, GWIN):
        pltpu.sync_copy(h_hbm.at[gidx.at[j]], rbufs[0])
        pltpu.sync_copy(rbufs[0], out_hbm.at[pl.ds(base + j * GW, GW)])


def _sage_layer(x, psum, pdeg, Ws, Wn, b, relu):
    BN = 1024

    def body(x_ref, p_ref, d_ref, ws_ref, wn_ref, b_ref, o_ref):
        p = p_ref[...]
        deg = jnp.maximum(jnp.sum(d_ref[...], axis=0), 1.0)[:, None]
        agg = (p[0] + p[1]) / deg
        y = _dot(x_ref[...], ws_ref[...]) + _dot(agg, wn_ref[...]) + b_ref[...]
        if relu:
            y = jnp.maximum(y, 0.0)
        o_ref[...] = y + x_ref[...]

    return pl.pallas_call(
        body,
        grid=(NPAD // BN,),
        in_specs=[
            pl.BlockSpec((BN, H), lambda i: (i, 0)),
            pl.BlockSpec((NC, BN, H), lambda i: (0, i, 0)),
            pl.BlockSpec((NW, BN), lambda i: (0, i)),
            pl.BlockSpec((D, H), lambda i: (0, 0)),
            pl.BlockSpec((D, H), lambda i: (0, 0)),
            pl.BlockSpec((1, H), lambda i: (0, 0)),
        ],
        out_specs=pl.BlockSpec((BN, H), lambda i: (i, 0)),
        out_shape=jax.ShapeDtypeStruct((N, H), _f32),
    )(x, psum, pdeg, Ws, Wn, b.reshape(1, H))


def _mlp_head(rows, W1, b1, W2, b2):
    BL = 512
    nblk = LPAD // BL

    def body(hs_ref, hd_ref, w1a_ref, w1b_ref, b1_ref, w2_ref, b2_ref, o_ref):
        z = _dot(hs_ref[...], w1a_ref[...]) + _dot(hd_ref[...], w1b_ref[...]) + b1_ref[...]
        z = jnp.maximum(z, 0.0)
        o_ref[...] = _dot(z, w2_ref[...]) + b2_ref[...]

    return pl.pallas_call(
        body,
        grid=(nblk,),
        in_specs=[
            pl.BlockSpec((BL, H), lambda i: (i, 0)),
            pl.BlockSpec((BL, H), lambda i, _n=nblk: (i + _n, 0)),
            pl.BlockSpec((H, H), lambda i: (0, 0)),
            pl.BlockSpec((H, H), lambda i: (0, 0)),
            pl.BlockSpec((1, H), lambda i: (0, 0)),
            pl.BlockSpec((H, 1), lambda i: (0, 0)),
            pl.BlockSpec((1, 1), lambda i: (0, 0)),
        ],
        out_specs=pl.BlockSpec((BL, 1), lambda i: (i, 0)),
        out_shape=jax.ShapeDtypeStruct((LPAD, 1), _f32),
    )(rows, rows, W1[:H], W1[H:], b1.reshape(1, H), W2, b2.reshape(1, 1))


def kernel(x, edge_index, edge_label_index, W_self_0, W_neigh_0, bias_0,
           W_self_1, W_neigh_1, bias_1, mlp_W1, mlp_b1, mlp_W2, mlp_b2):
    ep = E2 - E
    srcp = jnp.concatenate([edge_index[0], jnp.zeros((ep,), jnp.int32)])
    dstp = jnp.concatenate(
        [edge_index[1], N + (jnp.arange(ep, dtype=jnp.int32) % 128)])
    src3 = srcp.reshape(NW, NWIN, AW)
    dst3 = dstp.reshape(NW, NWIN, AW)
    zrow = jnp.zeros((NROW, H), _f32)
    zdeg = jnp.zeros((NPAD,), _f32)

    pdeg = _deg_hist(dst3, zdeg)
    psum0 = _seg_sum(x, src3, dst3, zrow)
    h1 = _sage_layer(x, psum0, pdeg, W_self_0, W_neigh_0, bias_0, relu=True)
    psum1 = _seg_sum(h1, src3, dst3, zrow)
    h2 = _sage_layer(h1, psum1, pdeg, W_self_1, W_neigh_1, bias_1, relu=False)

    pad = jnp.zeros((LPAD - L,), jnp.int32)
    gidx3 = jnp.concatenate(
        [edge_label_index[0], pad, edge_label_index[1], pad]).reshape(NW, GWIN, GW)
    rows = _gather_rows(h2, gidx3)
    out = _mlp_head(rows, mlp_W1, mlp_b1, mlp_W2, mlp_b2)
    return out[:L, 0]


# speedup vs baseline: 1.3983x; 1.3970x over previous
import functools

import jax
import jax.numpy as jnp
from jax import lax
from jax.experimental import pallas as pl
from jax.experimental.pallas import tpu as pltpu
from jax.experimental.pallas import tpu_sc as plsc

N = 10000
E = 320000
D = 128
H = 128
L = 100000

NC = 2
NS = 16
NW = NC * NS

EPW = E // NW
AW = 80
NWIN = EPW // AW
NPAD = 10240
NROW = NPAD // NS

LPAD = 100352
GW = 112
GWIN = (2 * LPAD) // NW // GW

_f32 = jnp.float32
_PH = lax.Precision.HIGHEST
_MESH = plsc.VectorSubcoreMesh(core_axis_name="c", subcore_axis_name="s")


def _dot(a, b):
    return lax.dot(a, b, precision=_PH, preferred_element_type=_f32)


def _deg_count(dst3):
    @functools.partial(
        pl.kernel, out_type=jax.ShapeDtypeStruct((NC, NPAD, H), _f32), mesh=_MESH,
        scratch_types=[
            pltpu.VMEM((AW,), jnp.int32),
            pltpu.VMEM((AW, H), _f32),
            pltpu.VMEM_SHARED((NPAD, H), _f32),
        ])
    def k(dst_hbm, zdeg_hbm, ones_hbm, pdeg_hbm, didx, onesv, accd):
        cid = lax.axis_index("c")
        sid = lax.axis_index("s")
        wid = sid * NC + cid
        r0 = sid * NROW
        pltpu.sync_copy(zdeg_hbm, accd.at[pl.ds(r0, NROW)])
        pltpu.sync_copy(ones_hbm, onesv)
        plsc.subcore_barrier()

        @pl.loop(0, NWIN)
        def _(j):
            pltpu.sync_copy(dst_hbm.at[wid, j], didx)
            pltpu.sync_copy(onesv, accd.at[didx], add=True)

        plsc.subcore_barrier()
        pltpu.sync_copy(accd.at[pl.ds(r0, NROW)], pdeg_hbm.at[cid, pl.ds(r0, NROW)])

    return k(dst3, jnp.zeros((NROW, H), _f32), jnp.ones((AW, H), _f32))


def _seg_sum(h, src3, dst3):
    @functools.partial(
        pl.kernel, out_type=jax.ShapeDtypeStruct((NC, NPAD, H), _f32), mesh=_MESH,
        scratch_types=[
            pltpu.VMEM((NWIN, AW), jnp.int32),
            pltpu.VMEM((AW,), jnp.int32),
            pltpu.VMEM((AW, H), _f32),
            pltpu.VMEM_SHARED((NPAD, H), _f32),
        ])
    def k(h_hbm, src_hbm, dst_hbm, zrow_hbm, psum_hbm, sidx, didx, rows, acc):
        cid = lax.axis_index("c")
        sid = lax.axis_index("s")
        wid = sid * NC + cid
        r0 = sid * NROW
        pltpu.sync_copy(zrow_hbm, acc.at[pl.ds(r0, NROW)])
        pltpu.sync_copy(src_hbm.at[wid], sidx)
        plsc.subcore_barrier()

        @pl.loop(0, NWIN)
        def _(j):
            pltpu.sync_copy(dst_hbm.at[wid, j], didx)
            pltpu.sync_copy(h_hbm.at[sidx.at[j]], rows)
            pltpu.sync_copy(rows, acc.at[didx], add=True)

        plsc.subcore_barrier()
        pltpu.sync_copy(acc.at[pl.ds(r0, NROW)], psum_hbm.at[cid, pl.ds(r0, NROW)])

    return k(h, src3, dst3, jnp.zeros((NROW, H), _f32))


def _gather_rows(h, gidx3):
    @functools.partial(
        pl.kernel, out_type=jax.ShapeDtypeStruct((2 * LPAD, H), _f32), mesh=_MESH,
        scratch_types=[
            pltpu.VMEM((GWIN, GW), jnp.int32),
            pltpu.VMEM((GW, H), _f32),
        ])
    def k(h_hbm, gidx_hbm, out_hbm, gidx, rows):
        cid = lax.axis_index("c")
        sid = lax.axis_index("s")
        wid = sid * NC + cid
        base = wid * (GWIN * GW)
        pltpu.sync_copy(gidx_hbm.at[wid], gidx)

        @pl.loop(0, GWIN)
        def _(j):
            pltpu.sync_copy(h_hbm.at[gidx.at[j]], rows)
            pltpu.sync_copy(rows, out_hbm.at[pl.ds(base + j * GW, GW)])

    return k(h, gidx3)


def _sage_layer(x, psum, pdeg, Ws, Wn, b, relu):
    BN = 1000

    def body(x_ref, p_ref, d_ref, ws_ref, wn_ref, b_ref, o_ref):
        p = p_ref[...]
        dd = d_ref[...]
        deg = jnp.maximum(dd[0, :, 0:1] + dd[1, :, 0:1], 1.0)
        agg = (p[0] + p[1]) / deg
        y = _dot(x_ref[...], ws_ref[...]) + _dot(agg, wn_ref[...]) + b_ref[...]
        if relu:
            y = jnp.maximum(y, 0.0)
        o_ref[...] = y + x_ref[...]

    return pl.pallas_call(
        body,
        grid=(N // BN,),
        in_specs=[
            pl.BlockSpec((BN, H), lambda i: (i, 0)),
            pl.BlockSpec((NC, BN, H), lambda i: (0, i, 0)),
            pl.BlockSpec((NC, BN, H), lambda i: (0, i, 0)),
            pl.BlockSpec((D, H), lambda i: (0, 0)),
            pl.BlockSpec((D, H), lambda i: (0, 0)),
            pl.BlockSpec((1, H), lambda i: (0, 0)),
        ],
        out_specs=pl.BlockSpec((BN, H), lambda i: (i, 0)),
        out_shape=jax.ShapeDtypeStruct((N, H), _f32),
    )(x, psum, pdeg, Ws, Wn, b.reshape(1, H))


def _mlp_head(rows, W1, b1, W2, b2):
    BL = 512
    nblk = LPAD // BL

    def body(hs_ref, hd_ref, w1a_ref, w1b_ref, b1_ref, w2_ref, b2_ref, o_ref):
        z = _dot(hs_ref[...], w1a_ref[...]) + _dot(hd_ref[...], w1b_ref[...]) + b1_ref[...]
        z = jnp.maximum(z, 0.0)
        o_ref[...] = _dot(z, w2_ref[...]) + b2_ref[...]

    return pl.pallas_call(
        body,
        grid=(nblk,),
        in_specs=[
            pl.BlockSpec((BL, H), lambda i: (i, 0)),
            pl.BlockSpec((BL, H), lambda i, _n=nblk: (i + _n, 0)),
            pl.BlockSpec((H, H), lambda i: (0, 0)),
            pl.BlockSpec((H, H), lambda i: (0, 0)),
            pl.BlockSpec((1, H), lambda i: (0, 0)),
            pl.BlockSpec((H, 1), lambda i: (0, 0)),
            pl.BlockSpec((1, 1), lambda i: (0, 0)),
        ],
        out_specs=pl.BlockSpec((BL, 1), lambda i: (i, 0)),
        out_shape=jax.ShapeDtypeStruct((LPAD, 1), _f32),
    )(rows, rows, W1[:H], W1[H:], b1.reshape(1, H), W2, b2.reshape(1, 1))


def kernel(x, edge_index, edge_label_index, W_self_0, W_neigh_0, bias_0,
           W_self_1, W_neigh_1, bias_1, mlp_W1, mlp_b1, mlp_W2, mlp_b2):
    src3 = edge_index[0].reshape(NW, NWIN, AW)
    dst3 = edge_index[1].reshape(NW, NWIN, AW)

    pdeg = _deg_count(dst3)
    psum0 = _seg_sum(x, src3, dst3)
    h1 = _sage_layer(x, psum0, pdeg, W_self_0, W_neigh_0, bias_0, relu=True)
    psum1 = _seg_sum(h1, src3, dst3)
    h2 = _sage_layer(h1, psum1, pdeg, W_self_1, W_neigh_1, bias_1, relu=False)

    pad = jnp.zeros((LPAD - L,), jnp.int32)
    gidx3 = jnp.concatenate(
        [edge_label_index[0], pad, edge_label_index[1], pad]).reshape(NW, GWIN, GW)
    rows = _gather_rows(h2, gidx3)
    out = _mlp_head(rows, mlp_W1, mlp_b1, mlp_W2, mlp_b2)
    return out[:L, 0]


# speedup vs baseline: 1.5782x; 1.1287x over previous
import functools

import dataclasses
import jax
import jax.numpy as jnp
from jax import lax
from jax.experimental import pallas as pl
from jax.experimental.pallas import tpu as pltpu
from jax.experimental.pallas import tpu_sc as plsc

N = 10000
E = 320000
D = 128
H = 128
L = 100000

NC = 2
NS = 16
NW = NC * NS

EPW = E // NW
AW = 80
NWIN = EPW // AW
NPAD = 10240
NROW = NPAD // NS

LPAD = 100352
GW = 112
GWIN = (2 * LPAD) // NW // GW

_f32 = jnp.float32
_PH = lax.Precision.HIGHEST
_MESH = plsc.VectorSubcoreMesh(core_axis_name="c", subcore_axis_name="s")

_CP = pltpu.CompilerParams()
if "needs_layout_passes" in pltpu.CompilerParams.__dataclass_fields__:
    _CP = dataclasses.replace(_CP, needs_layout_passes=False)


def _dot(a, b):
    return lax.dot(a, b, precision=_PH, preferred_element_type=_f32)


def _deg_count(dst3):
    @functools.partial(
        pl.kernel, out_type=jax.ShapeDtypeStruct((NW, NPAD), _f32), mesh=_MESH,
        compiler_params=_CP,
        scratch_types=[
            pltpu.VMEM((NWIN, AW), jnp.int32),
            pltpu.VMEM((NPAD,), _f32),
        ])
    def k(dst_hbm, zdeg_hbm, pdeg_hbm, didx, hist):
        cid = lax.axis_index("c")
        sid = lax.axis_index("s")
        wid = sid * NC + cid
        pltpu.sync_copy(dst_hbm.at[wid], didx)
        pltpu.sync_copy(zdeg_hbm, hist)
        ones16 = jnp.full((16,), 1.0, _f32)

        @pl.loop(0, NWIN)
        def _(j):
            for t in range(AW // 16):
                plsc.addupdate_scatter(hist, [didx[j, pl.ds(t * 16, 16)]], ones16)

        pltpu.sync_copy(hist, pdeg_hbm.at[wid])

    return k(dst3, jnp.zeros((NPAD,), _f32))


def _seg_sum(h, src3, dst3):
    @functools.partial(
        pl.kernel, out_type=jax.ShapeDtypeStruct((NC, NPAD, H), _f32), mesh=_MESH,
        scratch_types=[
            pltpu.VMEM((NWIN, AW), jnp.int32),
            pltpu.VMEM((AW,), jnp.int32),
            pltpu.VMEM((AW, H), _f32),
            pltpu.VMEM_SHARED((NPAD, H), _f32),
        ])
    def k(h_hbm, src_hbm, dst_hbm, zrow_hbm, psum_hbm, sidx, didx, rows, acc):
        cid = lax.axis_index("c")
        sid = lax.axis_index("s")
        wid = sid * NC + cid
        r0 = sid * NROW
        pltpu.sync_copy(zrow_hbm, acc.at[pl.ds(r0, NROW)])
        pltpu.sync_copy(src_hbm.at[wid], sidx)
        plsc.subcore_barrier()

        @pl.loop(0, NWIN)
        def _(j):
            pltpu.sync_copy(dst_hbm.at[wid, j], didx)
            pltpu.sync_copy(h_hbm.at[sidx.at[j]], rows)
            pltpu.sync_copy(rows, acc.at[didx], add=True)

        plsc.subcore_barrier()
        pltpu.sync_copy(acc.at[pl.ds(r0, NROW)], psum_hbm.at[cid, pl.ds(r0, NROW)])

    return k(h, src3, dst3, jnp.zeros((NROW, H), _f32))


def _gather_rows(h, gidx3):
    @functools.partial(
        pl.kernel, out_type=jax.ShapeDtypeStruct((2 * LPAD, H), _f32), mesh=_MESH,
        scratch_types=[
            pltpu.VMEM((GWIN, GW), jnp.int32),
            pltpu.VMEM((GW, H), _f32),
        ])
    def k(h_hbm, gidx_hbm, out_hbm, gidx, rows):
        cid = lax.axis_index("c")
        sid = lax.axis_index("s")
        wid = sid * NC + cid
        base = wid * (GWIN * GW)
        pltpu.sync_copy(gidx_hbm.at[wid], gidx)

        @pl.loop(0, GWIN)
        def _(j):
            pltpu.sync_copy(h_hbm.at[gidx.at[j]], rows)
            pltpu.sync_copy(rows, out_hbm.at[pl.ds(base + j * GW, GW)])

    return k(h, gidx3)


def _sage_layer(x, psum, pdeg, Ws, Wn, b, relu):
    BN = 1024

    def body(x_ref, p_ref, d_ref, ws_ref, wn_ref, b_ref, o_ref):
        p = p_ref[...]
        deg = jnp.maximum(jnp.sum(d_ref[...], axis=0), 1.0)[:, None]
        agg = (p[0] + p[1]) / deg
        y = _dot(x_ref[...], ws_ref[...]) + _dot(agg, wn_ref[...]) + b_ref[...]
        if relu:
            y = jnp.maximum(y, 0.0)
        o_ref[...] = y + x_ref[...]

    return pl.pallas_call(
        body,
        grid=(NPAD // BN,),
        in_specs=[
            pl.BlockSpec((BN, H), lambda i: (i, 0)),
            pl.BlockSpec((NC, BN, H), lambda i: (0, i, 0)),
            pl.BlockSpec((NW, BN), lambda i: (0, i)),
            pl.BlockSpec((D, H), lambda i: (0, 0)),
            pl.BlockSpec((D, H), lambda i: (0, 0)),
            pl.BlockSpec((1, H), lambda i: (0, 0)),
        ],
        out_specs=pl.BlockSpec((BN, H), lambda i: (i, 0)),
        out_shape=jax.ShapeDtypeStruct((N, H), _f32),
    )(x, psum, pdeg, Ws, Wn, b.reshape(1, H))


def _mlp_head(rows, W1, b1, W2, b2):
    BL = 512
    nblk = LPAD // BL

    def body(hs_ref, hd_ref, w1a_ref, w1b_ref, b1_ref, w2_ref, b2_ref, o_ref):
        z = _dot(hs_ref[...], w1a_ref[...]) + _dot(hd_ref[...], w1b_ref[...]) + b1_ref[...]
        z = jnp.maximum(z, 0.0)
        o_ref[...] = _dot(z, w2_ref[...]) + b2_ref[...]

    return pl.pallas_call(
        body,
        grid=(nblk,),
        in_specs=[
            pl.BlockSpec((BL, H), lambda i: (i, 0)),
            pl.BlockSpec((BL, H), lambda i, _n=nblk: (i + _n, 0)),
            pl.BlockSpec((H, H), lambda i: (0, 0)),
            pl.BlockSpec((H, H), lambda i: (0, 0)),
            pl.BlockSpec((1, H), lambda i: (0, 0)),
            pl.BlockSpec((H, 1), lambda i: (0, 0)),
            pl.BlockSpec((1, 1), lambda i: (0, 0)),
        ],
        out_specs=pl.BlockSpec((BL, 1), lambda i: (i, 0)),
        out_shape=jax.ShapeDtypeStruct((LPAD, 1), _f32),
    )(rows, rows, W1[:H], W1[H:], b1.reshape(1, H), W2, b2.reshape(1, 1))


def kernel(x, edge_index, edge_label_index, W_self_0, W_neigh_0, bias_0,
           W_self_1, W_neigh_1, bias_1, mlp_W1, mlp_b1, mlp_W2, mlp_b2):
    src3 = edge_index[0].reshape(NW, NWIN, AW)
    dst3 = edge_index[1].reshape(NW, NWIN, AW)

    pdeg = _deg_count(dst3)
    psum0 = _seg_sum(x, src3, dst3)
    h1 = _sage_layer(x, psum0, pdeg, W_self_0, W_neigh_0, bias_0, relu=True)
    psum1 = _seg_sum(h1, src3, dst3)
    h2 = _sage_layer(h1, psum1, pdeg, W_self_1, W_neigh_1, bias_1, relu=False)

    pad = jnp.zeros((LPAD - L,), jnp.int32)
    gidx3 = jnp.concatenate(
        [edge_label_index[0], pad, edge_label_index[1], pad]).reshape(NW, GWIN, GW)
    rows = _gather_rows(h2, gidx3)
    out = _mlp_head(rows, mlp_W1, mlp_b1, mlp_W2, mlp_b2)
    return out[:L, 0]


# speedup vs baseline: 1.6346x; 1.0357x over previous
import functools

import dataclasses
import jax
import jax.numpy as jnp
from jax import lax
from jax.experimental import pallas as pl
from jax.experimental.pallas import tpu as pltpu
from jax.experimental.pallas import tpu_sc as plsc

N = 10000
E = 320000
D = 128
H = 128
L = 100000

NC = 2
NS = 16
NW = NC * NS

EPW = E // NW
AW = 80
NWIN = EPW // AW
NPAD = 10240
NROW = NPAD // NS

LPAD = 100352
GW = 128
GWIN = (2 * LPAD) // NW // GW
NBUF = 4

_f32 = jnp.float32
_PH = lax.Precision.HIGHEST
_MESH = plsc.VectorSubcoreMesh(core_axis_name="c", subcore_axis_name="s")

_CP = pltpu.CompilerParams()
if "needs_layout_passes" in pltpu.CompilerParams.__dataclass_fields__:
    _CP = dataclasses.replace(_CP, needs_layout_passes=False)


def _dot(a, b):
    return lax.dot(a, b, precision=_PH, preferred_element_type=_f32)


def _deg_count(dst3):
    @functools.partial(
        pl.kernel, out_type=jax.ShapeDtypeStruct((NW, NPAD), _f32), mesh=_MESH,
        compiler_params=_CP,
        scratch_types=[
            pltpu.VMEM((NWIN, AW), jnp.int32),
            pltpu.VMEM((NPAD,), _f32),
        ])
    def k(dst_hbm, zdeg_hbm, pdeg_hbm, didx, hist):
        cid = lax.axis_index("c")
        sid = lax.axis_index("s")
        wid = sid * NC + cid
        pltpu.sync_copy(dst_hbm.at[wid], didx)
        pltpu.sync_copy(zdeg_hbm, hist)
        ones16 = jnp.full((16,), 1.0, _f32)

        @pl.loop(0, NWIN)
        def _(j):
            for t in range(AW // 16):
                plsc.addupdate_scatter(hist, [didx[j, pl.ds(t * 16, 16)]], ones16)

        pltpu.sync_copy(hist, pdeg_hbm.at[wid])

    return k(dst3, jnp.zeros((NPAD,), _f32))


def _seg_sum(h, src3, dst3):
    @functools.partial(
        pl.kernel, out_type=jax.ShapeDtypeStruct((NC, NPAD, H), _f32), mesh=_MESH,
        scratch_types=[
            pltpu.VMEM((NWIN, AW), jnp.int32),
            pltpu.VMEM((AW,), jnp.int32),
            pltpu.VMEM((AW, H), _f32),
            pltpu.VMEM_SHARED((NPAD, H), _f32),
        ])
    def k(h_hbm, src_hbm, dst_hbm, zrow_hbm, psum_hbm, sidx, didx, rows, acc):
        cid = lax.axis_index("c")
        sid = lax.axis_index("s")
        wid = sid * NC + cid
        r0 = sid * NROW
        pltpu.sync_copy(zrow_hbm, acc.at[pl.ds(r0, NROW)])
        pltpu.sync_copy(src_hbm.at[wid], sidx)
        plsc.subcore_barrier()

        @pl.loop(0, NWIN)
        def _(j):
            pltpu.sync_copy(dst_hbm.at[wid, j], didx)
            pltpu.sync_copy(h_hbm.at[sidx.at[j]], rows)
            pltpu.sync_copy(rows, acc.at[didx], add=True)

        plsc.subcore_barrier()
        pltpu.sync_copy(acc.at[pl.ds(r0, NROW)], psum_hbm.at[cid, pl.ds(r0, NROW)])

    return k(h, src3, dst3, jnp.zeros((NROW, H), _f32))


def _gather_rows(h, gidx3):
    @functools.partial(
        pl.kernel, out_type=jax.ShapeDtypeStruct((2 * LPAD, H), _f32), mesh=_MESH,
        scratch_types=([pltpu.VMEM((GWIN, GW), jnp.int32)]
                       + [pltpu.VMEM((GW, H), _f32) for _ in range(NBUF)]
                       + [pltpu.SemaphoreType.DMA for _ in range(2 * NBUF)]))
    def k(h_hbm, gidx_hbm, out_hbm, gidx, *refs):
        rbufs = refs[:NBUF]
        sems = refs[NBUF:]
        cid = lax.axis_index("c")
        sid = lax.axis_index("s")
        wid = sid * NC + cid
        base = wid * (GWIN * GW)
        pltpu.sync_copy(gidx_hbm.at[wid], gidx)
        main = (GWIN // NBUF) * NBUF

        @pl.loop(0, main, step=NBUF)
        def _(j):
            cgs = [pltpu.async_copy(h_hbm.at[gidx.at[j + b]], rbufs[b], sems[b])
                   for b in range(NBUF)]
            cws = []
            for b in range(NBUF):
                cgs[b].wait()
                cws.append(pltpu.async_copy(
                    rbufs[b], out_hbm.at[pl.ds(base + (j + b) * GW, GW)],
                    sems[NBUF + b]))
            for b in range(NBUF):
                cws[b].wait()

        for j in range(main, GWIN):
            pltpu.sync_copy(h_hbm.at[gidx.at[j]], rbufs[0])
            pltpu.sync_copy(rbufs[0], out_hbm.at[pl.ds(base + j * GW, GW)])

    return k(h, gidx3)


def _sage_layer(x, psum, pdeg, Ws, Wn, b, relu):
    BN = 1024

    def body(x_ref, p_ref, d_ref, ws_ref, wn_ref, b_ref, o_ref):
        p = p_ref[...]
        deg = jnp.maximum(jnp.sum(d_ref[...], axis=0), 1.0)[:, None]
        agg = (p[0] + p[1]) / deg
        y = _dot(x_ref[...], ws_ref[...]) + _dot(agg, wn_ref[...]) + b_ref[...]
        if relu:
            y = jnp.maximum(y, 0.0)
        o_ref[...] = y + x_ref[...]

    return pl.pallas_call(
        body,
        grid=(NPAD // BN,),
        in_specs=[
            pl.BlockSpec((BN, H), lambda i: (i, 0)),
            pl.BlockSpec((NC, BN, H), lambda i: (0, i, 0)),
            pl.BlockSpec((NW, BN), lambda i: (0, i)),
            pl.BlockSpec((D, H), lambda i: (0, 0)),
            pl.BlockSpec((D, H), lambda i: (0, 0)),
            pl.BlockSpec((1, H), lambda i: (0, 0)),
        ],
        out_specs=pl.BlockSpec((BN, H), lambda i: (i, 0)),
        out_shape=jax.ShapeDtypeStruct((N, H), _f32),
    )(x, psum, pdeg, Ws, Wn, b.reshape(1, H))


def _mlp_head(rows, W1, b1, W2, b2):
    BL = 512
    nblk = LPAD // BL

    def body(hs_ref, hd_ref, w1a_ref, w1b_ref, b1_ref, w2_ref, b2_ref, o_ref):
        z = _dot(hs_ref[...], w1a_ref[...]) + _dot(hd_ref[...], w1b_ref[...]) + b1_ref[...]
        z = jnp.maximum(z, 0.0)
        o_ref[...] = _dot(z, w2_ref[...]) + b2_ref[...]

    return pl.pallas_call(
        body,
        grid=(nblk,),
        in_specs=[
            pl.BlockSpec((BL, H), lambda i: (i, 0)),
            pl.BlockSpec((BL, H), lambda i, _n=nblk: (i + _n, 0)),
            pl.BlockSpec((H, H), lambda i: (0, 0)),
            pl.BlockSpec((H, H), lambda i: (0, 0)),
            pl.BlockSpec((1, H), lambda i: (0, 0)),
            pl.BlockSpec((H, 1), lambda i: (0, 0)),
            pl.BlockSpec((1, 1), lambda i: (0, 0)),
        ],
        out_specs=pl.BlockSpec((BL, 1), lambda i: (i, 0)),
        out_shape=jax.ShapeDtypeStruct((LPAD, 1), _f32),
    )(rows, rows, W1[:H], W1[H:], b1.reshape(1, H), W2, b2.reshape(1, 1))


def kernel(x, edge_index, edge_label_index, W_self_0, W_neigh_0, bias_0,
           W_self_1, W_neigh_1, bias_1, mlp_W1, mlp_b1, mlp_W2, mlp_b2):
    src3 = edge_index[0].reshape(NW, NWIN, AW)
    dst3 = edge_index[1].reshape(NW, NWIN, AW)

    pdeg = _deg_count(dst3)
    psum0 = _seg_sum(x, src3, dst3)
    h1 = _sage_layer(x, psum0, pdeg, W_self_0, W_neigh_0, bias_0, relu=True)
    psum1 = _seg_sum(h1, src3, dst3)
    h2 = _sage_layer(h1, psum1, pdeg, W_self_1, W_neigh_1, bias_1, relu=False)

    pad = jnp.zeros((LPAD - L,), jnp.int32)
    gidx3 = jnp.concatenate(
        [edge_label_index[0], pad, edge_label_index[1], pad]).reshape(NW, GWIN, GW)
    rows = _gather_rows(h2, gidx3)
    out = _mlp_head(rows, mlp_W1, mlp_b1, mlp_W2, mlp_b2)
    return out[:L, 0]


# speedup vs baseline: 2.0992x; 1.2842x over previous
import functools

import dataclasses
import jax
import jax.numpy as jnp
from jax import lax
from jax.experimental import pallas as pl
from jax.experimental.pallas import tpu as pltpu
from jax.experimental.pallas import tpu_sc as plsc

N = 10000
E = 320000
D = 128
H = 128
L = 100000

NC = 2
NS = 16
NW = NC * NS

EPW = E // NW
AW = 80
NWIN = EPW // AW
NPAD = 10240
NROW = NPAD // NS

LPAD = 100352
GW = 128
GWIN = (2 * LPAD) // NW // GW
NBUF = 4

_f32 = jnp.float32
_PH = lax.Precision.HIGHEST
_MESH = plsc.VectorSubcoreMesh(core_axis_name="c", subcore_axis_name="s")

_CP = pltpu.CompilerParams()
if "needs_layout_passes" in pltpu.CompilerParams.__dataclass_fields__:
    _CP = dataclasses.replace(_CP, needs_layout_passes=False)


def _dot(a, b):
    return lax.dot(a, b, precision=_PH, preferred_element_type=_f32)


def _deg_count(dst3):
    @functools.partial(
        pl.kernel, out_type=jax.ShapeDtypeStruct((NW, NPAD), _f32), mesh=_MESH,
        compiler_params=_CP,
        scratch_types=[
            pltpu.VMEM((NWIN, AW), jnp.int32),
            pltpu.VMEM((NPAD,), _f32),
        ])
    def k(dst_hbm, zdeg_hbm, pdeg_hbm, didx, hist):
        cid = lax.axis_index("c")
        sid = lax.axis_index("s")
        wid = sid * NC + cid
        pltpu.sync_copy(dst_hbm.at[wid], didx)
        pltpu.sync_copy(zdeg_hbm, hist)
        ones16 = jnp.full((16,), 1.0, _f32)

        @pl.loop(0, NWIN)
        def _(j):
            for t in range(AW // 16):
                plsc.addupdate_scatter(hist, [didx[j, pl.ds(t * 16, 16)]], ones16)

        pltpu.sync_copy(hist, pdeg_hbm.at[wid])

    return k(dst3, jnp.zeros((NPAD,), _f32))


SBUF = 3


def _seg_sum(h, src3, dst3):
    @functools.partial(
        pl.kernel, out_type=jax.ShapeDtypeStruct((NC, NPAD, H), _f32), mesh=_MESH,
        scratch_types=([pltpu.VMEM((NWIN, AW), jnp.int32)]
                       + [pltpu.VMEM((AW,), jnp.int32) for _ in range(SBUF)]
                       + [pltpu.VMEM((AW, H), _f32) for _ in range(SBUF)]
                       + [pltpu.VMEM_SHARED((NPAD, H), _f32)]
                       + [pltpu.SemaphoreType.DMA for _ in range(2 * SBUF)]))
    def k(h_hbm, src_hbm, dst_hbm, zrow_hbm, psum_hbm, *refs):
        sidx = refs[0]
        dbufs = refs[1:1 + SBUF]
        rbufs = refs[1 + SBUF:1 + 2 * SBUF]
        acc = refs[1 + 2 * SBUF]
        sems = refs[2 + 2 * SBUF:]
        cid = lax.axis_index("c")
        sid = lax.axis_index("s")
        wid = sid * NC + cid
        r0 = sid * NROW
        pltpu.sync_copy(zrow_hbm, acc.at[pl.ds(r0, NROW)])
        pltpu.sync_copy(src_hbm.at[wid], sidx)
        plsc.subcore_barrier()
        main = (NWIN // SBUF) * SBUF

        @pl.loop(0, main, step=SBUF)
        def _(j):
            cds = [pltpu.async_copy(dst_hbm.at[wid, j + b], dbufs[b], sems[b])
                   for b in range(SBUF)]
            cgs = [pltpu.async_copy(h_hbm.at[sidx.at[j + b]], rbufs[b],
                                    sems[SBUF + b])
                   for b in range(SBUF)]
            for b in range(SBUF):
                cds[b].wait()
                cgs[b].wait()
                pltpu.sync_copy(rbufs[b], acc.at[dbufs[b]], add=True)

        @pl.loop(main, NWIN)
        def _(j):
            pltpu.sync_copy(dst_hbm.at[wid, j], dbufs[0])
            pltpu.sync_copy(h_hbm.at[sidx.at[j]], rbufs[0])
            pltpu.sync_copy(rbufs[0], acc.at[dbufs[0]], add=True)

        plsc.subcore_barrier()
        pltpu.sync_copy(acc.at[pl.ds(r0, NROW)], psum_hbm.at[cid, pl.ds(r0, NROW)])

    return k(h, src3, dst3, jnp.zeros((NROW, H), _f32))


def _gather_rows(h, gidx3):
    @functools.partial(
        pl.kernel, out_type=jax.ShapeDtypeStruct((2 * LPAD, H), _f32), mesh=_MESH,
        scratch_types=([pltpu.VMEM((GWIN, GW), jnp.int32)]
                       + [pltpu.VMEM((GW, H), _f32) for _ in range(NBUF)]
                       + [pltpu.SemaphoreType.DMA for _ in range(2 * NBUF)]))
    def k(h_hbm, gidx_hbm, out_hbm, gidx, *refs):
        rbufs = refs[:NBUF]
        sems = refs[NBUF:]
        cid = lax.axis_index("c")
        sid = lax.axis_index("s")
        wid = sid * NC + cid
        base = wid * (GWIN * GW)
        pltpu.sync_copy(gidx_hbm.at[wid], gidx)
        main = (GWIN // NBUF) * NBUF

        @pl.loop(0, main, step=NBUF)
        def _(j):
            cgs = [pltpu.async_copy(h_hbm.at[gidx.at[j + b]], rbufs[b], sems[b])
                   for b in range(NBUF)]
            cws = []
            for b in range(NBUF):
                cgs[b].wait()
                cws.append(pltpu.async_copy(
                    rbufs[b], out_hbm.at[pl.ds(base + (j + b) * GW, GW)],
                    sems[NBUF + b]))
            for b in range(NBUF):
                cws[b].wait()

        for j in range(main, GWIN):
            pltpu.sync_copy(h_hbm.at[gidx.at[j]], rbufs[0])
            pltpu.sync_copy(rbufs[0], out_hbm.at[pl.ds(base + j * GW, GW)])

    return k(h, gidx3)


def _sage_layer(x, psum, pdeg, Ws, Wn, b, relu):
    BN = 1024

    def body(x_ref, p_ref, d_ref, ws_ref, wn_ref, b_ref, o_ref):
        p = p_ref[...]
        deg = jnp.maximum(jnp.sum(d_ref[...], axis=0), 1.0)[:, None]
        agg = (p[0] + p[1]) / deg
        y = _dot(x_ref[...], ws_ref[...]) + _dot(agg, wn_ref[...]) + b_ref[...]
        if relu:
            y = jnp.maximum(y, 0.0)
        o_ref[...] = y + x_ref[...]

    return pl.pallas_call(
        body,
        grid=(NPAD // BN,),
        in_specs=[
            pl.BlockSpec((BN, H), lambda i: (i, 0)),
            pl.BlockSpec((NC, BN, H), lambda i: (0, i, 0)),
            pl.BlockSpec((NW, BN), lambda i: (0, i)),
            pl.BlockSpec((D, H), lambda i: (0, 0)),
            pl.BlockSpec((D, H), lambda i: (0, 0)),
            pl.BlockSpec((1, H), lambda i: (0, 0)),
        ],
        out_specs=pl.BlockSpec((BN, H), lambda i: (i, 0)),
        out_shape=jax.ShapeDtypeStruct((N, H), _f32),
    )(x, psum, pdeg, Ws, Wn, b.reshape(1, H))


def _mlp_head(rows, W1, b1, W2, b2):
    BL = 512
    nblk = LPAD // BL

    def body(hs_ref, hd_ref, w1a_ref, w1b_ref, b1_ref, w2_ref, b2_ref, o_ref):
        z = _dot(hs_ref[...], w1a_ref[...]) + _dot(hd_ref[...], w1b_ref[...]) + b1_ref[...]
        z = jnp.maximum(z, 0.0)
        o_ref[...] = _dot(z, w2_ref[...]) + b2_ref[...]

    return pl.pallas_call(
        body,
        grid=(nblk,),
        in_specs=[
            pl.BlockSpec((BL, H), lambda i: (i, 0)),
            pl.BlockSpec((BL, H), lambda i, _n=nblk: (i + _n, 0)),
            pl.BlockSpec((H, H), lambda i: (0, 0)),
            pl.BlockSpec((H, H), lambda i: (0, 0)),
            pl.BlockSpec((1, H), lambda i: (0, 0)),
            pl.BlockSpec((H, 1), lambda i: (0, 0)),
            pl.BlockSpec((1, 1), lambda i: (0, 0)),
        ],
        out_specs=pl.BlockSpec((BL, 1), lambda i: (i, 0)),
        out_shape=jax.ShapeDtypeStruct((LPAD, 1), _f32),
    )(rows, rows, W1[:H], W1[H:], b1.reshape(1, H), W2, b2.reshape(1, 1))


def kernel(x, edge_index, edge_label_index, W_self_0, W_neigh_0, bias_0,
           W_self_1, W_neigh_1, bias_1, mlp_W1, mlp_b1, mlp_W2, mlp_b2):
    src3 = edge_index[0].reshape(NW, NWIN, AW)
    dst3 = edge_index[1].reshape(NW, NWIN, AW)

    pdeg = _deg_count(dst3)
    psum0 = _seg_sum(x, src3, dst3)
    h1 = _sage_layer(x, psum0, pdeg, W_self_0, W_neigh_0, bias_0, relu=True)
    psum1 = _seg_sum(h1, src3, dst3)
    h2 = _sage_layer(h1, psum1, pdeg, W_self_1, W_neigh_1, bias_1, relu=False)

    pad = jnp.zeros((LPAD - L,), jnp.int32)
    gidx3 = jnp.concatenate(
        [edge_label_index[0], pad, edge_label_index[1], pad]).reshape(NW, GWIN, GW)
    rows = _gather_rows(h2, gidx3)
    out = _mlp_head(rows, mlp_W1, mlp_b1, mlp_W2, mlp_b2)
    return out[:L, 0]


# speedup vs baseline: 2.3728x; 1.1303x over previous
import functools

import dataclasses
import jax
import jax.numpy as jnp
from jax import lax
from jax.experimental import pallas as pl
from jax.experimental.pallas import tpu as pltpu
from jax.experimental.pallas import tpu_sc as plsc

N = 10000
E = 320000
D = 128
H = 128
L = 100000

NC = 2
NS = 16
NW = NC * NS

EPW = E // NW
AW = 80
NWIN = EPW // AW
NPAD = 10240
NROW = NPAD // NS

LPAD = 100352
GW = 128
GWIN = (2 * LPAD) // NW // GW
NBUF = 6

_f32 = jnp.float32
_PH = lax.Precision.HIGHEST
_MESH = plsc.VectorSubcoreMesh(core_axis_name="c", subcore_axis_name="s")

_CP = pltpu.CompilerParams()
if "needs_layout_passes" in pltpu.CompilerParams.__dataclass_fields__:
    _CP = dataclasses.replace(_CP, needs_layout_passes=False)


def _dot(a, b):
    return lax.dot(a, b, precision=_PH, preferred_element_type=_f32)


def _deg_count(dst3):
    @functools.partial(
        pl.kernel, out_type=jax.ShapeDtypeStruct((NW, NPAD), _f32), mesh=_MESH,
        compiler_params=_CP,
        scratch_types=[
            pltpu.VMEM((NWIN, AW), jnp.int32),
            pltpu.VMEM((NPAD,), _f32),
        ])
    def k(dst_hbm, zdeg_hbm, pdeg_hbm, didx, hist):
        cid = lax.axis_index("c")
        sid = lax.axis_index("s")
        wid = sid * NC + cid
        pltpu.sync_copy(dst_hbm.at[wid], didx)
        pltpu.sync_copy(zdeg_hbm, hist)
        ones16 = jnp.full((16,), 1.0, _f32)

        @pl.loop(0, NWIN)
        def _(j):
            for t in range(AW // 16):
                plsc.addupdate_scatter(hist, [didx[j, pl.ds(t * 16, 16)]], ones16)

        pltpu.sync_copy(hist, pdeg_hbm.at[wid])

    return k(dst3, jnp.zeros((NPAD,), _f32))


SBUF = 3


def _seg_sum(h, src3, dst3):
    @functools.partial(
        pl.kernel, out_type=jax.ShapeDtypeStruct((NC, NPAD, H), _f32), mesh=_MESH,
        scratch_types=([pltpu.VMEM((NWIN, AW), jnp.int32)]
                       + [pltpu.VMEM((AW,), jnp.int32) for _ in range(SBUF)]
                       + [pltpu.VMEM((AW, H), _f32) for _ in range(SBUF)]
                       + [pltpu.VMEM_SHARED((NPAD, H), _f32)]
                       + [pltpu.SemaphoreType.DMA for _ in range(2 * SBUF)]))
    def k(h_hbm, src_hbm, dst_hbm, zrow_hbm, psum_hbm, *refs):
        sidx = refs[0]
        dbufs = refs[1:1 + SBUF]
        rbufs = refs[1 + SBUF:1 + 2 * SBUF]
        acc = refs[1 + 2 * SBUF]
        sems = refs[2 + 2 * SBUF:]
        cid = lax.axis_index("c")
        sid = lax.axis_index("s")
        wid = sid * NC + cid
        r0 = sid * NROW
        pltpu.sync_copy(zrow_hbm, acc.at[pl.ds(r0, NROW)])
        pltpu.sync_copy(src_hbm.at[wid], sidx)
        plsc.subcore_barrier()
        main = (NWIN // SBUF) * SBUF

        @pl.loop(0, main, step=SBUF)
        def _(j):
            cds = [pltpu.async_copy(dst_hbm.at[wid, j + b], dbufs[b], sems[b])
                   for b in range(SBUF)]
            cgs = [pltpu.async_copy(h_hbm.at[sidx.at[j + b]], rbufs[b],
                                    sems[SBUF + b])
                   for b in range(SBUF)]
            for b in range(SBUF):
                cds[b].wait()
                cgs[b].wait()
                pltpu.sync_copy(rbufs[b], acc.at[dbufs[b]], add=True)

        @pl.loop(main, NWIN)
        def _(j):
            pltpu.sync_copy(dst_hbm.at[wid, j], dbufs[0])
            pltpu.sync_copy(h_hbm.at[sidx.at[j]], rbufs[0])
            pltpu.sync_copy(rbufs[0], acc.at[dbufs[0]], add=True)

        plsc.subcore_barrier()
        pltpu.sync_copy(acc.at[pl.ds(r0, NROW)], psum_hbm.at[cid, pl.ds(r0, NROW)])

    return k(h, src3, dst3, jnp.zeros((NROW, H), _f32))


def _gather_rows(h, gidx3):
    @functools.partial(
        pl.kernel, out_type=jax.ShapeDtypeStruct((2 * LPAD, H), _f32), mesh=_MESH,
        scratch_types=([pltpu.VMEM((GWIN, GW), jnp.int32)]
                       + [pltpu.VMEM((GW, H), _f32) for _ in range(NBUF)]
                       + [pltpu.SemaphoreType.DMA for _ in range(2 * NBUF)]))
    def k(h_hbm, gidx_hbm, out_hbm, gidx, *refs):
        rbufs = refs[:NBUF]
        sems = refs[NBUF:]
        cid = lax.axis_index("c")
        sid = lax.axis_index("s")
        wid = sid * NC + cid
        base = wid * (GWIN * GW)
        pltpu.sync_copy(gidx_hbm.at[wid], gidx)
        main = (GWIN // NBUF) * NBUF

        @pl.loop(0, main, step=NBUF)
        def _(j):
            cgs = [pltpu.async_copy(h_hbm.at[gidx.at[j + b]], rbufs[b], sems[b])
                   for b in range(NBUF)]
            cws = []
            for b in range(NBUF):
                cgs[b].wait()
                cws.append(pltpu.async_copy(
                    rbufs[b], out_hbm.at[pl.ds(base + (j + b) * GW, GW)],
                    sems[NBUF + b]))
            for b in range(NBUF):
                cws[b].wait()

        for j in range(main, GWIN):
            pltpu.sync_copy(h_hbm.at[gidx.at[j]], rbufs[0])
            pltpu.sync_copy(rbufs[0], out_hbm.at[pl.ds(base + j * GW, GW)])

    return k(h, gidx3)


def _sage_layer(x, psum, pdeg, Ws, Wn, b, relu):
    BN = 1024

    def body(x_ref, p_ref, d_ref, ws_ref, wn_ref, b_ref, o_ref):
        p = p_ref[...]
        deg = jnp.maximum(jnp.sum(d_ref[...], axis=0), 1.0)[:, None]
        agg = (p[0] + p[1]) / deg
        y = _dot(x_ref[...], ws_ref[...]) + _dot(agg, wn_ref[...]) + b_ref[...]
        if relu:
            y = jnp.maximum(y, 0.0)
        o_ref[...] = y + x_ref[...]

    return pl.pallas_call(
        body,
        grid=(NPAD // BN,),
        in_specs=[
            pl.BlockSpec((BN, H), lambda i: (i, 0)),
            pl.BlockSpec((NC, BN, H), lambda i: (0, i, 0)),
            pl.BlockSpec((NW, BN), lambda i: (0, i)),
            pl.BlockSpec((D, H), lambda i: (0, 0)),
            pl.BlockSpec((D, H), lambda i: (0, 0)),
            pl.BlockSpec((1, H), lambda i: (0, 0)),
        ],
        out_specs=pl.BlockSpec((BN, H), lambda i: (i, 0)),
        out_shape=jax.ShapeDtypeStruct((N, H), _f32),
    )(x, psum, pdeg, Ws, Wn, b.reshape(1, H))


def _sage_layer_ab(x, psum, pdeg, Ws, Wn, b, W1a, W1b, b1):
    BN = 1024

    def body(x_ref, p_ref, d_ref, ws_ref, wn_ref, b_ref, w1a_ref, w1b_ref,
             b1_ref, a_ref, bb_ref):
        p = p_ref[...]
        deg = jnp.maximum(jnp.sum(d_ref[...], axis=0), 1.0)[:, None]
        agg = (p[0] + p[1]) / deg
        h2 = (_dot(x_ref[...], ws_ref[...]) + _dot(agg, wn_ref[...])
              + b_ref[...] + x_ref[...])
        a_ref[...] = _dot(h2, w1a_ref[...]) + b1_ref[...]
        bb_ref[...] = _dot(h2, w1b_ref[...])

    return pl.pallas_call(
        body,
        grid=(NPAD // BN,),
        in_specs=[
            pl.BlockSpec((BN, H), lambda i: (i, 0)),
            pl.BlockSpec((NC, BN, H), lambda i: (0, i, 0)),
            pl.BlockSpec((NW, BN), lambda i: (0, i)),
            pl.BlockSpec((D, H), lambda i: (0, 0)),
            pl.BlockSpec((D, H), lambda i: (0, 0)),
            pl.BlockSpec((1, H), lambda i: (0, 0)),
            pl.BlockSpec((H, H), lambda i: (0, 0)),
            pl.BlockSpec((H, H), lambda i: (0, 0)),
            pl.BlockSpec((1, H), lambda i: (0, 0)),
        ],
        out_specs=[pl.BlockSpec((BN, H), lambda i: (i, 0)),
                   pl.BlockSpec((BN, H), lambda i: (i, 0))],
        out_shape=[jax.ShapeDtypeStruct((N, H), _f32),
                   jax.ShapeDtypeStruct((N, H), _f32)],
    )(x, psum, pdeg, Ws, Wn, b.reshape(1, H), W1a, W1b, b1.reshape(1, H))


def _mlp_head(rows, W2, b2):
    BL = 512
    nblk = LPAD // BL

    def body(hs_ref, hd_ref, w2_ref, b2_ref, o_ref):
        z = jnp.maximum(hs_ref[...] + hd_ref[...], 0.0)
        o_ref[...] = jnp.sum(z * w2_ref[...], axis=1, keepdims=True) + b2_ref[...]

    return pl.pallas_call(
        body,
        grid=(nblk,),
        in_specs=[
            pl.BlockSpec((BL, H), lambda i: (i, 0)),
            pl.BlockSpec((BL, H), lambda i, _n=nblk: (i + _n, 0)),
            pl.BlockSpec((1, H), lambda i: (0, 0)),
            pl.BlockSpec((1, 1), lambda i: (0, 0)),
        ],
        out_specs=pl.BlockSpec((BL, 1), lambda i: (i, 0)),
        out_shape=jax.ShapeDtypeStruct((LPAD, 1), _f32),
    )(rows, rows, W2.reshape(1, H), b2.reshape(1, 1))


def _mlp_head_old(rows, W1, b1, W2, b2):
    BL = 512
    nblk = LPAD // BL

    def body(hs_ref, hd_ref, w1a_ref, w1b_ref, b1_ref, w2_ref, b2_ref, o_ref):
        z = _dot(hs_ref[...], w1a_ref[...]) + _dot(hd_ref[...], w1b_ref[...]) + b1_ref[...]
        z = jnp.maximum(z, 0.0)
        o_ref[...] = _dot(z, w2_ref[...]) + b2_ref[...]

    return pl.pallas_call(
        body,
        grid=(nblk,),
        in_specs=[
            pl.BlockSpec((BL, H), lambda i: (i, 0)),
            pl.BlockSpec((BL, H), lambda i, _n=nblk: (i + _n, 0)),
            pl.BlockSpec((H, H), lambda i: (0, 0)),
            pl.BlockSpec((H, H), lambda i: (0, 0)),
            pl.BlockSpec((1, H), lambda i: (0, 0)),
            pl.BlockSpec((H, 1), lambda i: (0, 0)),
            pl.BlockSpec((1, 1), lambda i: (0, 0)),
        ],
        out_specs=pl.BlockSpec((BL, 1), lambda i: (i, 0)),
        out_shape=jax.ShapeDtypeStruct((LPAD, 1), _f32),
    )(rows, rows, W1[:H], W1[H:], b1.reshape(1, H), W2, b2.reshape(1, 1))


def kernel(x, edge_index, edge_label_index, W_self_0, W_neigh_0, bias_0,
           W_self_1, W_neigh_1, bias_1, mlp_W1, mlp_b1, mlp_W2, mlp_b2):
    src3 = edge_index[0].reshape(NW, NWIN, AW)
    dst3 = edge_index[1].reshape(NW, NWIN, AW)

    pdeg = _deg_count(dst3)
    psum0 = _seg_sum(x, src3, dst3)
    h1 = _sage_layer(x, psum0, pdeg, W_self_0, W_neigh_0, bias_0, relu=True)
    psum1 = _seg_sum(h1, src3, dst3)
    A, B = _sage_layer_ab(h1, psum1, pdeg, W_self_1, W_neigh_1, bias_1,
                          mlp_W1[:H], mlp_W1[H:], mlp_b1)
    AB = jnp.concatenate([A, B], axis=0)

    pad = jnp.zeros((LPAD - L,), jnp.int32)
    gidx3 = jnp.concatenate(
        [edge_label_index[0], pad, edge_label_index[1] + N, pad]).reshape(NW, GWIN, GW)
    rows = _gather_rows(AB, gidx3)
    out = _mlp_head(rows, mlp_W2, mlp_b2)
    return out[:L, 0]


# speedup vs baseline: 2.3749x; 1.0009x over previous
import functools

import dataclasses
import jax
import jax.numpy as jnp
from jax import lax
from jax.experimental import pallas as pl
from jax.experimental.pallas import tpu as pltpu
from jax.experimental.pallas import tpu_sc as plsc

N = 10000
E = 320000
D = 128
H = 128
L = 100000

NC = 2
NS = 16
NW = NC * NS

EPW = E // NW
AW = 80
NWIN = EPW // AW
NPAD = 10240
NROW = NPAD // NS

LPAD = 100352
GW = 128
GWIN = (2 * LPAD) // NW // GW
NBUF = 6

_f32 = jnp.float32
_PH = lax.Precision.HIGHEST
_MESH = plsc.VectorSubcoreMesh(core_axis_name="c", subcore_axis_name="s")

_CP = pltpu.CompilerParams()
if "needs_layout_passes" in pltpu.CompilerParams.__dataclass_fields__:
    _CP = dataclasses.replace(_CP, needs_layout_passes=False)


def _dot(a, b):
    return lax.dot(a, b, precision=_PH, preferred_element_type=_f32)


def _deg_count(dst3):
    @functools.partial(
        pl.kernel, out_type=jax.ShapeDtypeStruct((NW, NPAD), _f32), mesh=_MESH,
        compiler_params=_CP,
        scratch_types=[
            pltpu.VMEM((NWIN, AW), jnp.int32),
            pltpu.VMEM((NPAD,), _f32),
        ])
    def k(dst_hbm, zdeg_hbm, pdeg_hbm, didx, hist):
        cid = lax.axis_index("c")
        sid = lax.axis_index("s")
        wid = sid * NC + cid
        pltpu.sync_copy(dst_hbm.at[wid], didx)
        pltpu.sync_copy(zdeg_hbm, hist)
        ones16 = jnp.full((16,), 1.0, _f32)

        @pl.loop(0, NWIN)
        def _(j):
            for t in range(AW // 16):
                plsc.addupdate_scatter(hist, [didx[j, pl.ds(t * 16, 16)]], ones16)

        pltpu.sync_copy(hist, pdeg_hbm.at[wid])

    return k(dst3, jnp.zeros((NPAD,), _f32))


SBUF = 3


def _seg_sum(h, src3, dst3):
    @functools.partial(
        pl.kernel, out_type=jax.ShapeDtypeStruct((NC, NPAD, H), _f32), mesh=_MESH,
        scratch_types=([pltpu.VMEM((NWIN, AW), jnp.int32)]
                       + [pltpu.VMEM((AW,), jnp.int32) for _ in range(SBUF)]
                       + [pltpu.VMEM((AW, H), _f32) for _ in range(SBUF)]
                       + [pltpu.VMEM_SHARED((NPAD, H), _f32)]
                       + [pltpu.SemaphoreType.DMA for _ in range(2 * SBUF)]))
    def k(h_hbm, src_hbm, dst_hbm, zrow_hbm, psum_hbm, *refs):
        sidx = refs[0]
        dbufs = refs[1:1 + SBUF]
        rbufs = refs[1 + SBUF:1 + 2 * SBUF]
        acc = refs[1 + 2 * SBUF]
        sems = refs[2 + 2 * SBUF:]
        cid = lax.axis_index("c")
        sid = lax.axis_index("s")
        wid = sid * NC + cid
        r0 = sid * NROW
        pltpu.sync_copy(zrow_hbm, acc.at[pl.ds(r0, NROW)])
        pltpu.sync_copy(src_hbm.at[wid], sidx)
        plsc.subcore_barrier()
        main = (NWIN // SBUF) * SBUF

        @pl.loop(0, main, step=SBUF)
        def _(j):
            cds = [pltpu.async_copy(dst_hbm.at[wid, j + b], dbufs[b], sems[b])
                   for b in range(SBUF)]
            cgs = [pltpu.async_copy(h_hbm.at[sidx.at[j + b]], rbufs[b],
                                    sems[SBUF + b])
                   for b in range(SBUF)]
            for b in range(SBUF):
                cds[b].wait()
                cgs[b].wait()
                pltpu.sync_copy(rbufs[b], acc.at[dbufs[b]], add=True)

        @pl.loop(main, NWIN)
        def _(j):
            pltpu.sync_copy(dst_hbm.at[wid, j], dbufs[0])
            pltpu.sync_copy(h_hbm.at[sidx.at[j]], rbufs[0])
            pltpu.sync_copy(rbufs[0], acc.at[dbufs[0]], add=True)

        plsc.subcore_barrier()
        pltpu.sync_copy(acc.at[pl.ds(r0, NROW)], psum_hbm.at[cid, pl.ds(r0, NROW)])

    return k(h, src3, dst3, jnp.zeros((NROW, H), _f32))


def _gather_rows(h, gidx3):
    @functools.partial(
        pl.kernel, out_type=jax.ShapeDtypeStruct((2 * LPAD, H), _f32), mesh=_MESH,
        scratch_types=([pltpu.VMEM((GWIN, GW), jnp.int32)]
                       + [pltpu.VMEM((GW, H), _f32) for _ in range(NBUF)]
                       + [pltpu.SemaphoreType.DMA for _ in range(2 * NBUF)]))
    def k(h_hbm, gidx_hbm, out_hbm, gidx, *refs):
        rbufs = refs[:NBUF]
        sems = refs[NBUF:]
        cid = lax.axis_index("c")
        sid = lax.axis_index("s")
        wid = sid * NC + cid
        base = wid * (GWIN * GW)
        pltpu.sync_copy(gidx_hbm.at[wid], gidx)
        main = (GWIN // NBUF) * NBUF

        @pl.loop(0, main, step=NBUF)
        def _(j):
            cgs = [pltpu.async_copy(h_hbm.at[gidx.at[j + b]], rbufs[b], sems[b])
                   for b in range(NBUF)]
            cws = []
            for b in range(NBUF):
                cgs[b].wait()
                cws.append(pltpu.async_copy(
                    rbufs[b], out_hbm.at[pl.ds(base + (j + b) * GW, GW)],
                    sems[NBUF + b]))
            for b in range(NBUF):
                cws[b].wait()

        for j in range(main, GWIN):
            pltpu.sync_copy(h_hbm.at[gidx.at[j]], rbufs[0])
            pltpu.sync_copy(rbufs[0], out_hbm.at[pl.ds(base + j * GW, GW)])

    return k(h, gidx3)


def _sage_layer(x, psum, pdeg, Ws, Wn, b, relu):
    BN = 1024

    def body(x_ref, p_ref, d_ref, ws_ref, wn_ref, b_ref, o_ref):
        p = p_ref[...]
        deg = jnp.maximum(jnp.sum(d_ref[...], axis=0), 1.0)[:, None]
        agg = (p[0] + p[1]) / deg
        y = _dot(x_ref[...], ws_ref[...]) + _dot(agg, wn_ref[...]) + b_ref[...]
        if relu:
            y = jnp.maximum(y, 0.0)
        o_ref[...] = y + x_ref[...]

    return pl.pallas_call(
        body,
        grid=(NPAD // BN,),
        in_specs=[
            pl.BlockSpec((BN, H), lambda i: (i, 0)),
            pl.BlockSpec((NC, BN, H), lambda i: (0, i, 0)),
            pl.BlockSpec((NW, BN), lambda i: (0, i)),
            pl.BlockSpec((D, H), lambda i: (0, 0)),
            pl.BlockSpec((D, H), lambda i: (0, 0)),
            pl.BlockSpec((1, H), lambda i: (0, 0)),
        ],
        out_specs=pl.BlockSpec((BN, H), lambda i: (i, 0)),
        out_shape=jax.ShapeDtypeStruct((N, H), _f32),
    )(x, psum, pdeg, Ws, Wn, b.reshape(1, H))


def _sage_layer_ab(x, psum, pdeg, Ws, Wn, b, W1a, W1b, b1):
    BN = 1024

    def body(x_ref, p_ref, d_ref, ws_ref, wn_ref, b_ref, w1a_ref, w1b_ref,
             b1_ref, a_ref, bb_ref):
        p = p_ref[...]
        deg = jnp.maximum(jnp.sum(d_ref[...], axis=0), 1.0)[:, None]
        agg = (p[0] + p[1]) / deg
        h2 = (_dot(x_ref[...], ws_ref[...]) + _dot(agg, wn_ref[...])
              + b_ref[...] + x_ref[...])
        a_ref[...] = _dot(h2, w1a_ref[...]) + b1_ref[...]
        bb_ref[...] = _dot(h2, w1b_ref[...])

    return pl.pallas_call(
        body,
        grid=(NPAD // BN,),
        in_specs=[
            pl.BlockSpec((BN, H), lambda i: (i, 0)),
            pl.BlockSpec((NC, BN, H), lambda i: (0, i, 0)),
            pl.BlockSpec((NW, BN), lambda i: (0, i)),
            pl.BlockSpec((D, H), lambda i: (0, 0)),
            pl.BlockSpec((D, H), lambda i: (0, 0)),
            pl.BlockSpec((1, H), lambda i: (0, 0)),
            pl.BlockSpec((H, H), lambda i: (0, 0)),
            pl.BlockSpec((H, H), lambda i: (0, 0)),
            pl.BlockSpec((1, H), lambda i: (0, 0)),
        ],
        out_specs=[pl.BlockSpec((BN, H), lambda i: (i, 0)),
                   pl.BlockSpec((BN, H), lambda i: (i, 0))],
        out_shape=[jax.ShapeDtypeStruct((N, H), _f32),
                   jax.ShapeDtypeStruct((N, H), _f32)],
    )(x, psum, pdeg, Ws, Wn, b.reshape(1, H), W1a, W1b, b1.reshape(1, H))


def _mlp_head(rows, W2, b2):
    BL = 512
    nblk = LPAD // BL

    def body(hs_ref, hd_ref, w2_ref, b2_ref, o_ref):
        z = jnp.maximum(hs_ref[...] + hd_ref[...], 0.0)
        o_ref[...] = jnp.sum(z * w2_ref[...], axis=1, keepdims=True) + b2_ref[...]

    return pl.pallas_call(
        body,
        grid=(nblk,),
        in_specs=[
            pl.BlockSpec((BL, H), lambda i: (i, 0)),
            pl.BlockSpec((BL, H), lambda i, _n=nblk: (i + _n, 0)),
            pl.BlockSpec((1, H), lambda i: (0, 0)),
            pl.BlockSpec((1, 1), lambda i: (0, 0)),
        ],
        out_specs=pl.BlockSpec((BL, 1), lambda i: (i, 0)),
        out_shape=jax.ShapeDtypeStruct((L, 1), _f32),
    )(rows, rows, W2.reshape(1, H), b2.reshape(1, 1))


def _mlp_head_old(rows, W1, b1, W2, b2):
    BL = 512
    nblk = LPAD // BL

    def body(hs_ref, hd_ref, w1a_ref, w1b_ref, b1_ref, w2_ref, b2_ref, o_ref):
        z = _dot(hs_ref[...], w1a_ref[...]) + _dot(hd_ref[...], w1b_ref[...]) + b1_ref[...]
        z = jnp.maximum(z, 0.0)
        o_ref[...] = _dot(z, w2_ref[...]) + b2_ref[...]

    return pl.pallas_call(
        body,
        grid=(nblk,),
        in_specs=[
            pl.BlockSpec((BL, H), lambda i: (i, 0)),
            pl.BlockSpec((BL, H), lambda i, _n=nblk: (i + _n, 0)),
            pl.BlockSpec((H, H), lambda i: (0, 0)),
            pl.BlockSpec((H, H), lambda i: (0, 0)),
            pl.BlockSpec((1, H), lambda i: (0, 0)),
            pl.BlockSpec((H, 1), lambda i: (0, 0)),
            pl.BlockSpec((1, 1), lambda i: (0, 0)),
        ],
        out_specs=pl.BlockSpec((BL, 1), lambda i: (i, 0)),
        out_shape=jax.ShapeDtypeStruct((LPAD, 1), _f32),
    )(rows, rows, W1[:H], W1[H:], b1.reshape(1, H), W2, b2.reshape(1, 1))


def kernel(x, edge_index, edge_label_index, W_self_0, W_neigh_0, bias_0,
           W_self_1, W_neigh_1, bias_1, mlp_W1, mlp_b1, mlp_W2, mlp_b2):
    src3 = edge_index[0].reshape(NW, NWIN, AW)
    dst3 = edge_index[1].reshape(NW, NWIN, AW)

    pdeg = _deg_count(dst3)
    psum0 = _seg_sum(x, src3, dst3)
    h1 = _sage_layer(x, psum0, pdeg, W_self_0, W_neigh_0, bias_0, relu=True)
    psum1 = _seg_sum(h1, src3, dst3)
    A, B = _sage_layer_ab(h1, psum1, pdeg, W_self_1, W_neigh_1, bias_1,
                          mlp_W1[:H], mlp_W1[H:], mlp_b1)
    AB = jnp.concatenate([A, B], axis=0)

    pad = jnp.zeros((LPAD - L,), jnp.int32)
    gidx3 = jnp.concatenate(
        [edge_label_index[0], pad, edge_label_index[1] + N, pad]).reshape(NW, GWIN, GW)
    rows = _gather_rows(AB, gidx3)
    out = _mlp_head(rows, mlp_W2, mlp_b2)
    return out[:, 0]
